# Initial kernel scaffold; baseline (speedup 1.0000x reference)
import jax
import jax.numpy as jnp
from jax.experimental import pallas as pl


def kernel(icdcode, embed_table):
    raise NotImplementedError("write your pallas kernel here")


# speedup vs baseline: 11.4321x; 11.4321x over previous
import functools

import jax
import jax.numpy as jnp
from jax import lax
from jax.experimental import pallas as pl
from jax.experimental.pallas import tpu as pltpu
from jax.experimental.pallas import tpu_sc as plsc

B = 16384
L = 50
D = 50
DP = 56
LANES = 16

_info = plsc.get_sparse_core_info()
NC, NS = _info.num_cores, _info.num_subcores
NW = NC * NS
SPW = B // NW
C = 16
NCHUNK = SPW // C
RPC = C * L
GB = 80
NGB = RPC // GB

_mesh = plsc.VectorSubcoreMesh(core_axis_name="c", subcore_axis_name="s")


@functools.partial(
    pl.kernel,
    out_type=jax.ShapeDtypeStruct((B, DP), jnp.float32),
    mesh=_mesh,
    scratch_types=[
        pltpu.VMEM((NGB, GB), jnp.int32),
        pltpu.VMEM((RPC, DP), jnp.float32),
        pltpu.VMEM((C, DP), jnp.float32),
        pltpu.SemaphoreType.DMA,
    ],
    compiler_params=pltpu.CompilerParams(use_tc_tiling_on_sc=False),
)
def _gram_mean(idx_hbm, table_hbm, out_hbm, idx_v, rows_v, out_v, sem):
    wid = lax.axis_index("s") * NC + lax.axis_index("c")
    scale = jnp.float32(1.0 / L)

    @pl.loop(0, NCHUNK)
    def _chunk(c):
        sbase = wid * SPW + c * C
        pltpu.sync_copy(idx_hbm.at[pl.ds(sbase * L // GB, NGB)], idx_v)
        copies = [
            pltpu.async_copy(
                table_hbm.at[idx_v.at[k]],
                rows_v.at[pl.ds(k * GB, GB)],
                sem,
            )
            for k in range(NGB)
        ]
        for cp in copies:
            cp.wait()

        @pl.loop(0, C)
        def _sample(s):
            def rbody(r, accs):
                row = s * L + r
                return (
                    accs[0] + rows_v[row, pl.ds(0, LANES)],
                    accs[1] + rows_v[row, pl.ds(16, LANES)],
                    accs[2] + rows_v[row, pl.ds(32, LANES)],
                    accs[3] + rows_v[row, pl.ds(40, LANES)],
                )

            z = jnp.zeros((LANES,), jnp.float32)
            a0, a1, a2, a3 = lax.fori_loop(0, L, rbody, (z, z, z, z))
            out_v[s, pl.ds(0, LANES)] = a0 * scale
            out_v[s, pl.ds(16, LANES)] = a1 * scale
            out_v[s, pl.ds(32, LANES)] = a2 * scale
            out_v[s, pl.ds(40, LANES)] = a3 * scale

        pltpu.sync_copy(out_v, out_hbm.at[pl.ds(sbase, C)])


def kernel(icdcode, embed_table):
    idx_2d = icdcode.reshape(B * L // GB, GB).astype(jnp.int32)
    table_p = jnp.pad(embed_table.astype(jnp.float32), ((0, 0), (0, DP - D)))
    out_p = _gram_mean(idx_2d, table_p)
    return out_p[:, :D]


# speedup vs baseline: 12.2721x; 1.0735x over previous
import functools

import jax
import jax.numpy as jnp
from jax import lax
from jax.experimental import pallas as pl
from jax.experimental.pallas import tpu as pltpu
from jax.experimental.pallas import tpu_sc as plsc

B = 16384
L = 50
D = 50
DP = 56
LANES = 16

_info = plsc.get_sparse_core_info()
NC, NS = _info.num_cores, _info.num_subcores
NW = NC * NS
SPW = B // NW
C = 16
NCHUNK = SPW // C
RPC = C * L
GB = 80
NGB = RPC // GB

_mesh = plsc.VectorSubcoreMesh(core_axis_name="c", subcore_axis_name="s")


@functools.partial(
    pl.kernel,
    out_type=jax.ShapeDtypeStruct((B, DP), jnp.float32),
    mesh=_mesh,
    scratch_types=[
        pltpu.VMEM((NGB, GB), jnp.int32),
        pltpu.VMEM((RPC, DP), jnp.float32),
        pltpu.VMEM((C, DP), jnp.float32),
        pltpu.SemaphoreType.DMA,
    ],
    compiler_params=pltpu.CompilerParams(use_tc_tiling_on_sc=False),
)
def _gram_mean(idx_hbm, table_hbm, out_hbm, idx_v, rows_v, out_v, sem):
    wid = lax.axis_index("s") * NC + lax.axis_index("c")
    scale = jnp.float32(1.0 / L)

    @pl.loop(0, NCHUNK)
    def _chunk(c):
        sbase = wid * SPW + c * C
        pltpu.sync_copy(idx_hbm.at[pl.ds(sbase * L // GB, NGB)], idx_v)
        copies = [
            pltpu.async_copy(
                table_hbm.at[idx_v.at[k]],
                rows_v.at[pl.ds(k * GB, GB)],
                sem,
            )
            for k in range(NGB)
        ]
        for cp in copies:
            cp.wait()

        @pl.loop(0, C)
        def _sample(s):
            base = s * L
            z = jnp.zeros((LANES,), jnp.float32)
            a0, a1, a2, a3 = z, z, z, z
            for r in range(L):
                a0 = a0 + rows_v[base + r, pl.ds(0, LANES)]
                a1 = a1 + rows_v[base + r, pl.ds(16, LANES)]
                a2 = a2 + rows_v[base + r, pl.ds(32, LANES)]
                a3 = a3 + rows_v[base + r, pl.ds(40, LANES)]
            out_v[s, pl.ds(0, LANES)] = a0 * scale
            out_v[s, pl.ds(16, LANES)] = a1 * scale
            out_v[s, pl.ds(32, LANES)] = a2 * scale
            out_v[s, pl.ds(40, LANES)] = a3 * scale

        pltpu.sync_copy(out_v, out_hbm.at[pl.ds(sbase, C)])


def kernel(icdcode, embed_table):
    idx_2d = icdcode.reshape(B * L // GB, GB).astype(jnp.int32)
    table_p = jnp.pad(embed_table.astype(jnp.float32), ((0, 0), (0, DP - D)))
    out_p = _gram_mean(idx_2d, table_p)
    return out_p[:, :D]


# speedup vs baseline: 16.7792x; 1.3673x over previous
import functools

import jax
import jax.numpy as jnp
from jax import lax
from jax.experimental import pallas as pl
from jax.experimental.pallas import tpu as pltpu
from jax.experimental.pallas import tpu_sc as plsc

B = 16384
L = 50
D = 50
DP = 56
LANES = 16

_info = plsc.get_sparse_core_info()
NC, NS = _info.num_cores, _info.num_subcores
NW = NC * NS
SPW = B // NW
C = 16
NCHUNK = SPW // C
RPC = C * L
GB = 80
NGB = RPC // GB

_mesh = plsc.VectorSubcoreMesh(core_axis_name="c", subcore_axis_name="s")


NIDX = NCHUNK * NGB


@functools.partial(
    pl.kernel,
    out_type=jax.ShapeDtypeStruct((B, DP), jnp.float32),
    mesh=_mesh,
    scratch_types=[
        pltpu.VMEM((NIDX, GB), jnp.int32),
        pltpu.VMEM((RPC, DP), jnp.float32),
        pltpu.VMEM((RPC, DP), jnp.float32),
        pltpu.VMEM((C, DP), jnp.float32),
        pltpu.SemaphoreType.DMA,
        pltpu.SemaphoreType.DMA,
    ],
    compiler_params=pltpu.CompilerParams(use_tc_tiling_on_sc=False),
)
def _gram_mean(idx_hbm, table_hbm, out_hbm, idx_v, rows0, rows1, out_v,
               sem0, sem1):
    wid = lax.axis_index("s") * NC + lax.axis_index("c")
    scale = jnp.float32(1.0 / L)

    pltpu.sync_copy(idx_hbm.at[pl.ds(wid * NIDX, NIDX)], idx_v)

    def fire(c, buf, sem):
        for k in range(NGB):
            pltpu.async_copy(
                table_hbm.at[idx_v.at[c * NGB + k]],
                buf.at[pl.ds(k * GB, GB)],
                sem,
            )

    def drain(buf, sem):
        pltpu.make_async_copy(table_hbm.at[pl.ds(0, RPC)], buf, sem).wait()

    fire(0, rows0, sem0)

    @pl.loop(0, NCHUNK, step=2)
    def _pair(c0):
        for b in range(2):
            c = c0 + b
            cur, csem = (rows0, sem0) if b == 0 else (rows1, sem1)
            nxt, nsem = (rows1, sem1) if b == 0 else (rows0, sem0)

            @pl.when(c + 1 < NCHUNK)
            def _():
                fire(c + 1, nxt, nsem)

            drain(cur, csem)

            @pl.loop(0, C)
            def _sample(s):
                base = s * L
                z = jnp.zeros((LANES,), jnp.float32)
                a0, a1, a2, a3 = z, z, z, z
                for r in range(L):
                    a0 = a0 + cur[base + r, pl.ds(0, LANES)]
                    a1 = a1 + cur[base + r, pl.ds(16, LANES)]
                    a2 = a2 + cur[base + r, pl.ds(32, LANES)]
                    a3 = a3 + cur[base + r, pl.ds(40, LANES)]
                out_v[s, pl.ds(0, LANES)] = a0 * scale
                out_v[s, pl.ds(16, LANES)] = a1 * scale
                out_v[s, pl.ds(32, LANES)] = a2 * scale
                out_v[s, pl.ds(40, LANES)] = a3 * scale

            pltpu.sync_copy(out_v, out_hbm.at[pl.ds(wid * SPW + c * C, C)])


def kernel(icdcode, embed_table):
    idx_2d = icdcode.reshape(B * L // GB, GB).astype(jnp.int32)
    table_p = jnp.pad(embed_table.astype(jnp.float32), ((0, 0), (0, DP - D)))
    out_p = _gram_mean(idx_2d, table_p)
    return out_p[:, :D]
